# Initial kernel scaffold; baseline (speedup 1.0000x reference)
import jax
import jax.numpy as jnp
from jax.experimental import pallas as pl


def kernel(x, pos_embedding):
    raise NotImplementedError("write your pallas kernel here")


# speedup vs baseline: 8.1632x; 8.1632x over previous
import jax
import jax.numpy as jnp
from jax.experimental import pallas as pl

S = 8192
N = 4
D = 768
BS = 1024


def _bcast_kernel(emb_ref, out_ref):
    out_ref[...] = jnp.broadcast_to(emb_ref[...][:, None, :], (BS, N, D))


def kernel(x, pos_embedding):
    del x
    return pl.pallas_call(
        _bcast_kernel,
        grid=(S // BS,),
        in_specs=[pl.BlockSpec((BS, D), lambda i: (i, 0))],
        out_specs=pl.BlockSpec((BS, N, D), lambda i: (i, 0, 0)),
        out_shape=jax.ShapeDtypeStruct((S, N, D), jnp.float32),
    )(pos_embedding)
